# baseline (speedup 1.0000x reference)
import functools

import jax
import jax.numpy as jnp
from jax import lax
from jax.experimental import pallas as pl
from jax.experimental.pallas import tpu as pltpu
from jax.experimental.pallas import tpu_sc as plsc

VOCAB = 1000
SEQ = 200
WORDS = SEQ // 2
BATCH = 4096
NCLS = 4
TROWS = 1008
PITCH = 105


def _proj_body(emb_ref, w_ref, b_ref, out_ref):
    t = lax.dot_general(
        emb_ref[:], w_ref[:],
        dimension_numbers=(((1,), (1,)), ((), ())),
        preferred_element_type=jnp.float32,
    )
    out_ref[0:VOCAB, :] = t * (1.0 / SEQ)
    out_ref[VOCAB:TROWS, :] = b_ref[:]


def _project_table(emb_table, W, b):
    bpad = jnp.zeros((TROWS - VOCAB, NCLS), jnp.float32).at[0].set(b)
    return pl.pallas_call(
        _proj_body,
        out_shape=jax.ShapeDtypeStruct((TROWS, NCLS), jnp.float32),
    )(emb_table, W, bpad)


def _pack_pairs(T):
    tb = T.astype(jnp.bfloat16).reshape(TROWS, 2, 2)
    w = lax.bitcast_convert_type(tb, jnp.int32)
    return jnp.concatenate([w[:, 0], w[:, 1]])


def _pack_indices(x):
    w = lax.bitcast_convert_type(
        x.astype(jnp.int16).reshape(BATCH, WORDS, 2), jnp.int32)
    return jnp.pad(w, ((0, 0), (0, PITCH - WORDS)))


def _pool_softmax(t2, xw):
    info = plsc.get_sparse_core_info()
    nc, ns, L = info.num_cores, info.num_subcores, info.num_lanes
    nw = nc * ns
    b_per_w = BATCH // nw
    groups = b_per_w // L
    mesh = plsc.VectorSubcoreMesh(core_axis_name="c", subcore_axis_name="s")

    def unpack_f32(acc):
        u = plsc.bitcast(acc, jnp.int32)
        himask = jnp.full((L,), -65536, jnp.int32)
        lo = plsc.bitcast(lax.shift_left(u, 16), jnp.float32)
        hi = plsc.bitcast(jnp.bitwise_and(u, himask), jnp.float32)
        return lo, hi

    @functools.partial(
        pl.kernel,
        mesh=mesh,
        compiler_params=pltpu.CompilerParams(
            needs_layout_passes=False, use_tc_tiling_on_sc=False),
        out_type=jax.ShapeDtypeStruct((BATCH, NCLS), jnp.float32),
        scratch_types=[
            pltpu.VMEM((2 * TROWS,), jnp.int32),
            pltpu.VMEM((2, L, PITCH), jnp.int32),
            pltpu.VMEM((b_per_w, NCLS), jnp.float32),
            pltpu.SemaphoreType.DMA,
            pltpu.SemaphoreType.DMA,
        ],
    )
    def k(t_hbm, x_hbm, out_hbm, t_v, xg, o_v, sem0, sem1):
        wid = lax.axis_index("s") * nc + lax.axis_index("c")
        base = wid * b_per_w
        sems = (sem0, sem1)

        def start_fetch(g):
            return pltpu.async_copy(
                x_hbm.at[pl.ds(base + g * L, L)],
                xg.at[g % 2],
                sems[g % 2],
            )

        pending = start_fetch(0)
        pltpu.sync_copy(t_hbm, t_v)

        iota = lax.iota(jnp.int32, L)
        bias_lo = jnp.full((L,), VOCAB, jnp.int32)
        bias_hi = jnp.full((L,), TROWS + VOCAB, jnp.int32)
        lomask = jnp.full((L,), 0xFFFF, jnp.int32)
        cols = [jnp.full((L,), c, jnp.int32) for c in range(NCLS)]

        for g in range(groups):
            pending.wait()
            if g + 1 < groups:
                pending = start_fetch(g + 1)

            buf = jnp.full((L,), g % 2, jnp.int32)
            acc_a = plsc.bitcast(plsc.load_gather(t_v, [bias_lo]),
                                 jnp.bfloat16)
            acc_b = plsc.bitcast(plsc.load_gather(t_v, [bias_hi]),
                                 jnp.bfloat16)

            def step(w, accs):
                aa, ab = accs
                wv = jnp.broadcast_to(w, (L,)).astype(jnp.int32)
                pair = plsc.load_gather(xg, [buf, iota, wv])
                i0 = jnp.bitwise_and(pair, lomask)
                i1 = lax.shift_right_logical(pair, 16)
                aa = aa + plsc.bitcast(plsc.load_gather(t_v, [i0]),
                                       jnp.bfloat16)
                ab = ab + plsc.bitcast(plsc.load_gather(t_v, [i0 + TROWS]),
                                       jnp.bfloat16)
                aa = aa + plsc.bitcast(plsc.load_gather(t_v, [i1]),
                                       jnp.bfloat16)
                ab = ab + plsc.bitcast(plsc.load_gather(t_v, [i1 + TROWS]),
                                       jnp.bfloat16)
                return (aa, ab)

            acc_a, acc_b = lax.fori_loop(0, WORDS, step, (acc_a, acc_b),
                                         unroll=2)

            a0, a1 = unpack_f32(acc_a)
            a2, a3 = unpack_f32(acc_b)
            m = jnp.maximum(jnp.maximum(a0, a1), jnp.maximum(a2, a3))
            e0 = jnp.exp(a0 - m)
            e1 = jnp.exp(a1 - m)
            e2 = jnp.exp(a2 - m)
            e3 = jnp.exp(a3 - m)
            s = (e0 + e1) + (e2 + e3)
            rows = g * L + iota
            for c, ec in enumerate((e0, e1, e2, e3)):
                plsc.store_scatter(o_v, [rows, cols[c]], ec / s)

        pltpu.sync_copy(o_v, out_hbm.at[pl.ds(base, b_per_w)])

    return k(t2, xw)


def kernel(x, emb_table, W, b):
    t2 = _pack_pairs(_project_table(emb_table, W, b))
    return _pool_softmax(t2, _pack_indices(x))


# speedup vs baseline: 1.2101x; 1.2101x over previous
import functools

import jax
import jax.numpy as jnp
from jax import lax
from jax.experimental import pallas as pl
from jax.experimental.pallas import tpu as pltpu
from jax.experimental.pallas import tpu_sc as plsc

VOCAB = 1000
SEQ = 200
BATCH = 4096
NCLS = 4
TROWS = 1008
ZROW = VOCAB + 1


def _proj_body(emb_ref, w_ref, b_ref, out_ref):
    t = lax.dot_general(
        emb_ref[:], w_ref[:],
        dimension_numbers=(((1,), (1,)), ((), ())),
        preferred_element_type=jnp.float32,
    )
    out_ref[0:VOCAB, :] = t * (1.0 / SEQ)
    out_ref[VOCAB:TROWS, :] = b_ref[:]


def _project_table(emb_table, W, b):
    bpad = jnp.zeros((TROWS - VOCAB, NCLS), jnp.float32).at[0].set(b)
    return pl.pallas_call(
        _proj_body,
        out_shape=jax.ShapeDtypeStruct((TROWS, NCLS), jnp.float32),
    )(emb_table, W, bpad)


def _pack_pairs(T):
    tb = T.astype(jnp.bfloat16).reshape(TROWS, 2, 2)
    w = lax.bitcast_convert_type(tb, jnp.int32)
    return jnp.concatenate([w[:, 0], w[:, 1]])


def _pool_softmax(t2, x):
    info = plsc.get_sparse_core_info()
    nc, ns, L = info.num_cores, info.num_subcores, info.num_lanes
    nw = nc * ns
    b_per_w = BATCH // nw
    groups = b_per_w // L
    chunks = SEQ // L
    tail = SEQ - chunks * L
    mesh = plsc.VectorSubcoreMesh(core_axis_name="c", subcore_axis_name="s")

    def unpack_f32(acc):
        u = plsc.bitcast(acc, jnp.int32)
        himask = jnp.full((L,), -65536, jnp.int32)
        lo = plsc.bitcast(lax.shift_left(u, 16), jnp.float32)
        hi = plsc.bitcast(jnp.bitwise_and(u, himask), jnp.float32)
        return lo, hi

    @functools.partial(
        pl.kernel,
        mesh=mesh,
        compiler_params=pltpu.CompilerParams(needs_layout_passes=False),
        out_type=jax.ShapeDtypeStruct((BATCH, NCLS), jnp.float32),
        scratch_types=[
            pltpu.VMEM((2 * TROWS,), jnp.int32),
            pltpu.VMEM((2, L, SEQ), jnp.int32),
            pltpu.VMEM((b_per_w, NCLS), jnp.float32),
            pltpu.SemaphoreType.DMA,
            pltpu.SemaphoreType.DMA,
        ],
    )
    def k(t_hbm, x_hbm, out_hbm, t_v, xg, o_v, sem0, sem1):
        wid = lax.axis_index("s") * nc + lax.axis_index("c")
        base = wid * b_per_w
        sems = (sem0, sem1)

        def start_fetch(g):
            return pltpu.async_copy(
                x_hbm.at[pl.ds(base + g * L, L)],
                xg.at[g % 2],
                sems[g % 2],
            )

        pending = start_fetch(0)
        pltpu.sync_copy(t_hbm, t_v)

        iota = lax.iota(jnp.int32, L)
        zero = jnp.zeros((L,), jnp.float32)
        tailmask = iota < tail
        zrow = jnp.full((L,), ZROW, jnp.int32)
        cols = [jnp.full((L,), c, jnp.int32) for c in range(NCLS)]

        blo = plsc.load_gather(t_v, [jnp.full((L,), VOCAB, jnp.int32)])
        bhi = plsc.load_gather(t_v,
                               [jnp.full((L,), TROWS + VOCAB, jnp.int32)])
        b0, b1 = unpack_f32(plsc.bitcast(blo, jnp.bfloat16))
        b2, b3 = unpack_f32(plsc.bitcast(bhi, jnp.bfloat16))

        for g in range(groups):
            pending.wait()
            if g + 1 < groups:
                pending = start_fetch(g + 1)

            bufv = jnp.full((L,), g % 2, jnp.int32)

            def row_step(r, stage):
                s0, s1, s2, s3 = stage
                rv = jnp.broadcast_to(r, (L,)).astype(jnp.int32)
                acc_a = plsc.bitcast(jnp.zeros((L,), jnp.int32),
                                     jnp.bfloat16)
                acc_b = acc_a
                for c in range(chunks + (1 if tail else 0)):
                    pos = jnp.full((L,), c * L, jnp.int32) + iota
                    if c >= chunks:
                        pos = jnp.where(tailmask, pos, iota)
                    idx = plsc.load_gather(xg, [bufv, rv, pos])
                    if c >= chunks:
                        idx = jnp.where(tailmask, idx, zrow)
                    glo = plsc.load_gather(t_v, [idx])
                    ghi = plsc.load_gather(t_v, [idx + TROWS])
                    acc_a = acc_a + plsc.bitcast(glo, jnp.bfloat16)
                    acc_b = acc_b + plsc.bitcast(ghi, jnp.bfloat16)
                a0, a1 = unpack_f32(acc_a)
                a2, a3 = unpack_f32(acc_b)
                mask = iota == jnp.broadcast_to(r, (L,)).astype(jnp.int32)
                s0 = s0 + jnp.where(mask, jnp.broadcast_to(
                    jnp.sum(a0, axis=0), (L,)), zero)
                s1 = s1 + jnp.where(mask, jnp.broadcast_to(
                    jnp.sum(a1, axis=0), (L,)), zero)
                s2 = s2 + jnp.where(mask, jnp.broadcast_to(
                    jnp.sum(a2, axis=0), (L,)), zero)
                s3 = s3 + jnp.where(mask, jnp.broadcast_to(
                    jnp.sum(a3, axis=0), (L,)), zero)
                return (s0, s1, s2, s3)

            a0, a1, a2, a3 = lax.fori_loop(
                0, L, row_step, (b0, b1, b2, b3))

            m = jnp.maximum(jnp.maximum(a0, a1), jnp.maximum(a2, a3))
            e0 = jnp.exp(a0 - m)
            e1 = jnp.exp(a1 - m)
            e2 = jnp.exp(a2 - m)
            e3 = jnp.exp(a3 - m)
            s = (e0 + e1) + (e2 + e3)
            rows = g * L + iota
            for c, ec in enumerate((e0, e1, e2, e3)):
                plsc.store_scatter(o_v, [rows, cols[c]], ec / s)

        pltpu.sync_copy(o_v, out_hbm.at[pl.ds(base, b_per_w)])

    return k(t2, x)


def kernel(x, emb_table, W, b):
    t2 = _pack_pairs(_project_table(emb_table, W, b))
    return _pool_softmax(t2, x)
